# baseline (speedup 1.0000x reference)
import functools

import jax
import jax.numpy as jnp
from jax import lax
from jax.experimental import pallas as pl
from jax.experimental.pallas import tpu as pltpu
from jax.experimental.pallas import tpu_sc as plsc

N = 2048
DIM = 512
NS = 256
C = 16
KP1 = 12
BR = 256
NB = N // BR
BIGF = 3.0e38

_info = plsc.get_sparse_core_info()
_NC, _NSC, _L = _info.num_cores, _info.num_subcores, _info.num_lanes
NW = _NC * _NSC
RPW = N // NW
NCH = N // 16


def _a_body(fs_ref, ys_ref, fq_ref, dist_ref, a_ref, lam_ref, y_ref,
            q2_ref, q2t_ref):
    eye_b = (lax.broadcasted_iota(jnp.int32, (BR, BR), 0)
             == lax.broadcasted_iota(jnp.int32, (BR, BR), 1)).astype(jnp.float32)

    def q2_blk(bi, _):
        fq = fq_ref[pl.ds(bi * BR, BR), :]
        q2_ref[pl.ds(bi * BR, BR), :] = jnp.sum(fq * fq, axis=1, keepdims=True)
        return 0
    lax.fori_loop(0, NB, q2_blk, 0)

    def t_blk(bi, _):
        q2t_ref[:, pl.ds(bi * BR, BR)] = lax.dot_general(
            q2_ref[pl.ds(bi * BR, BR), :], eye_b, (((0,), (0,)), ((), ())),
            preferred_element_type=jnp.float32, precision=lax.Precision.HIGHEST)
        return 0
    lax.fori_loop(0, NB, t_blk, 0)

    cls_iota = lax.broadcasted_iota(jnp.int32, (C, NS), 0)
    onehot = (cls_iota == ys_ref[...]).astype(jnp.float32)
    counts = jnp.sum(onehot, axis=1, keepdims=True)
    protos = lax.dot_general(
        onehot, fs_ref[...], (((1,), (0,)), ((), ())),
        preferred_element_type=jnp.float32,
        precision=lax.Precision.HIGHEST) / counts

    p2 = jnp.sum(protos * protos, axis=1, keepdims=True)
    eye16 = (lax.broadcasted_iota(jnp.int32, (C, C), 0)
             == lax.broadcasted_iota(jnp.int32, (C, C), 1)).astype(jnp.float32)
    p2t = lax.dot_general(p2, eye16, (((0,), (0,)), ((), ())),
                          preferred_element_type=jnp.float32,
                          precision=lax.Precision.HIGHEST)
    pq = lax.dot_general(fq_ref[...], protos, (((1,), (1,)), ((), ())),
                         preferred_element_type=jnp.float32)
    d2p = jnp.maximum(q2_ref[...] + p2t - 2.0 * pq, 0.0)
    distp = jnp.sqrt(d2p + 1e-12)
    a_ref[...] = distp * distp
    d = jnp.min(distp, axis=1, keepdims=True)

    db = lax.bitcast_convert_type(d, jnp.int32)
    def med_step(_, c):
        lo, hi = c
        mid = lo + (hi - lo) // 2
        cnt = jnp.sum((db <= mid).astype(jnp.int32))
        take = cnt >= (N // 2)
        return jnp.where(take, lo, mid + 1), jnp.where(take, mid, hi)
    lo, _hi = lax.fori_loop(0, 31, med_step, (jnp.int32(0), jnp.int32(2**31 - 1)))
    med = lax.bitcast_convert_type(lo, jnp.float32)
    lam_ref[...] = jnp.exp(-(d * d) / (2.0 * med * med + 1e-8))

    na = -a_ref[...]
    mx = jnp.max(na, axis=1, keepdims=True)
    e = jnp.exp(na - mx)
    y_ref[...] = e / jnp.sum(e, axis=1, keepdims=True)

    def dist_blk(bi, _):
        fq = fq_ref[pl.ds(bi * BR, BR), :]
        g = lax.dot_general(fq, fq_ref[...], (((1,), (1,)), ((), ())),
                            preferred_element_type=jnp.float32)
        d2 = jnp.maximum(q2_ref[pl.ds(bi * BR, BR), :] + q2t_ref[...] - 2.0 * g, 0.0)
        dist_ref[pl.ds(bi * BR, BR), :] = jnp.sqrt(d2 + 1e-12)
        return 0
    lax.fori_loop(0, NB, dist_blk, 0)


def _sc_topk(dist):
    mesh = plsc.VectorSubcoreMesh(core_axis_name="c", subcore_axis_name="s")

    @functools.partial(
        pl.kernel, mesh=mesh,
        out_type=(jax.ShapeDtypeStruct((N * N,), jnp.float32),
                  jax.ShapeDtypeStruct((N * 16,), jnp.float32)),
        scratch_types=[
            pltpu.VMEM((N,), jnp.float32),
            pltpu.VMEM((N,), jnp.float32),
            pltpu.VMEM((NCH,), jnp.float32),
        ],
        compiler_params=pltpu.CompilerParams(needs_layout_passes=False),
    )
    def k(dist_hbm, wraw_hbm, sig_hbm, row_v, line_v, cm_v):
        wid = lax.axis_index("s") * _NC + lax.axis_index("c")
        base = wid * RPW
        lane = lax.iota(jnp.int32, 16)
        lane0 = lane == 0
        zeros16 = jnp.zeros((16,), jnp.float32)
        big16 = jnp.full((16,), BIGF, jnp.float32)

        def row_body(r, _):
            gr = base + r
            pltpu.sync_copy(dist_hbm.at[pl.ds(gr * N, N)], row_v)

            def zl(c, _):
                line_v[pl.ds(c * 16, 16)] = zeros16
                return 0
            lax.fori_loop(0, NCH, zl, 0)

            def cmb(c, _):
                v = row_v[pl.ds(c * 16, 16)]
                mn = lax.reduce_min(v, axes=(0,))
                plsc.store_scatter(cm_v, [jnp.broadcast_to(c, (16,))],
                                   jnp.broadcast_to(mn, (16,)), mask=lane0)
                return 0
            lax.fori_loop(0, NCH, cmb, 0)

            def round_body(j, _):
                def gm(cc, acc):
                    return jnp.minimum(acc, cm_v[pl.ds(cc * 16, 16)])
                accv = lax.fori_loop(0, NCH // 16, gm, big16)
                mval = lax.reduce_min(accv, axes=(0,))

                def fc(cc, st):
                    slot, ln = st
                    eq = cm_v[pl.ds(cc * 16, 16)] == mval
                    cand = jnp.where(eq, lane, jnp.int32(16))
                    l_s = lax.reduce_min(cand, axes=(0,))
                    take = (slot < 0) & (l_s < 16)
                    return (jnp.where(take, cc, slot),
                            jnp.where(take, l_s, ln))
                cslot, clane = lax.fori_loop(
                    0, NCH // 16, fc, (jnp.int32(-1), jnp.int32(0)))
                ci = cslot * 16 + clane

                v = plsc.load_gather(row_v, [jnp.broadcast_to(ci * 16, (16,)) + lane])
                candl = jnp.where(v == mval, lane, jnp.int32(16))
                el = lax.reduce_min(candl, axes=(0,))
                col = ci * 16 + el

                plsc.store_scatter(row_v, [jnp.broadcast_to(col, (16,))],
                                   big16, mask=lane0)
                v2 = plsc.load_gather(row_v, [jnp.broadcast_to(ci * 16, (16,)) + lane])
                mn2 = lax.reduce_min(v2, axes=(0,))
                plsc.store_scatter(cm_v, [jnp.broadcast_to(ci, (16,))],
                                   jnp.broadcast_to(mn2, (16,)), mask=lane0)

                @pl.when(j > 0)
                def _():
                    plsc.store_scatter(line_v, [jnp.broadcast_to(col, (16,))],
                                       jnp.broadcast_to(mval, (16,)), mask=lane0)
                return jnp.broadcast_to(mval, (16,))
            last = lax.fori_loop(0, KP1, round_body, zeros16)

            pltpu.sync_copy(line_v, wraw_hbm.at[pl.ds(gr * N, N)])
            line_v[pl.ds(0, 16)] = last + 1e-8
            pltpu.sync_copy(line_v.at[pl.ds(0, 16)], sig_hbm.at[pl.ds(gr * 16, 16)])
            return 0
        lax.fori_loop(0, RPW, row_body, 0)

    return k(dist)


def _b_body(wraw_ref, sig_ref, a_ref, lam_ref, y0_ref, out_ref,
            wx_ref, sigt_ref, y_ref, coef_ref):
    eye_b = (lax.broadcasted_iota(jnp.int32, (BR, BR), 0)
             == lax.broadcasted_iota(jnp.int32, (BR, BR), 1)).astype(jnp.float32)

    def t_blk(bi, _):
        sigt_ref[:, pl.ds(bi * BR, BR)] = lax.dot_general(
            sig_ref[pl.ds(bi * BR, BR), :], eye_b, (((0,), (0,)), ((), ())),
            preferred_element_type=jnp.float32, precision=lax.Precision.HIGHEST)
        return 0
    lax.fori_loop(0, NB, t_blk, 0)

    y_ref[...] = y0_ref[...]

    def wexp_blk(bi, _):
        sl = pl.ds(bi * BR, BR)
        wv = wraw_ref[sl, :]
        sig = sig_ref[sl, :]
        wx_ref[sl, :] = jnp.where(wv > 0.0,
                                  jnp.exp(-wv / (sig * sigt_ref[...])), 0.0)
        return 0
    lax.fori_loop(0, NB, wexp_blk, 0)

    def w_bi(bi, _):
        def w_bj(bj, _):
            su = pl.ds(bi * BR, BR)
            sv = pl.ds(bj * BR, BR)
            blk_a = wx_ref[su, sv]
            blk_b = wx_ref[sv, su]
            w1 = (blk_a + jnp.transpose(blk_b)) / 2.0
            wx_ref[su, sv] = w1
            wx_ref[sv, su] = jnp.transpose(w1)
            return 0
        lax.fori_loop(bi, NB, w_bj, 0)
        return 0
    lax.fori_loop(0, NB, w_bi, 0)

    def coef_blk(bi, _):
        sl = pl.ds(bi * BR, BR)
        rs = jnp.sum(wx_ref[sl, :], axis=1, keepdims=True)
        d_inv = 1.0 / (rs + 1e-8)
        coef_ref[sl, :] = lam_ref[sl, :] * d_inv
        return 0
    lax.fori_loop(0, NB, coef_blk, 0)

    neg_a = -a_ref[...]
    coef = coef_ref[...]

    def lp_cond(c):
        i, stop = c
        return (i < 50) & jnp.logical_not(stop)

    def lp_body(c):
        i, stop = c
        y_old = y_ref[...]
        z = lax.dot_general(wx_ref[...], y_old, (((1,), (0,)), ((), ())),
                            preferred_element_type=jnp.float32)
        logits = neg_a + coef * z
        mxl = jnp.max(logits, axis=1, keepdims=True)
        el = jnp.exp(logits - mxl)
        y_new = el / jnp.sum(el, axis=1, keepdims=True)
        converged = jnp.max(jnp.abs(y_new - y_old)) < 1e-4

        @pl.when(jnp.logical_not(converged))
        def _():
            y_ref[...] = y_new
        return i + 1, converged

    lax.while_loop(lp_cond, lp_body, (jnp.int32(0), jnp.bool_(False)))

    y = y_ref[...]
    mxy = jnp.max(y, axis=1, keepdims=True)
    cols = lax.broadcasted_iota(jnp.int32, (N, C), 1)
    out_ref[...] = jnp.min(jnp.where(y == mxy, cols, C), axis=1, keepdims=True)


def kernel(feat_s, y_s, feat_q):
    ys2d = y_s.astype(jnp.int32).reshape(1, NS)

    dist, a, lam, y0 = pl.pallas_call(
        _a_body,
        out_shape=(jax.ShapeDtypeStruct((N, N), jnp.float32),
                   jax.ShapeDtypeStruct((N, C), jnp.float32),
                   jax.ShapeDtypeStruct((N, 1), jnp.float32),
                   jax.ShapeDtypeStruct((N, C), jnp.float32)),
        scratch_shapes=[
            pltpu.VMEM((N, 1), jnp.float32),
            pltpu.VMEM((1, N), jnp.float32),
        ],
    )(feat_s, ys2d, feat_q)

    wraw1d, sig1d = _sc_topk(dist.reshape(N * N))
    wraw = wraw1d.reshape(N, N)
    sig = sig1d.reshape(N, 16)[:, 0:1]

    out = pl.pallas_call(
        _b_body,
        out_shape=jax.ShapeDtypeStruct((N, 1), jnp.int32),
        scratch_shapes=[
            pltpu.VMEM((N, N), jnp.float32),
            pltpu.VMEM((1, N), jnp.float32),
            pltpu.VMEM((N, C), jnp.float32),
            pltpu.VMEM((N, 1), jnp.float32),
        ],
    )(wraw, sig, a, lam, y0)
    return out.reshape(N)


# speedup vs baseline: 1.1182x; 1.1182x over previous
import functools

import jax
import jax.numpy as jnp
from jax import lax
from jax.experimental import pallas as pl
from jax.experimental.pallas import tpu as pltpu
from jax.experimental.pallas import tpu_sc as plsc

N = 2048
DIM = 512
NS = 256
C = 16
KP1 = 12
BR = 256
NB = N // BR
BIGF = 3.0e38

_info = plsc.get_sparse_core_info()
_NC, _NSC, _L = _info.num_cores, _info.num_subcores, _info.num_lanes
NW = _NC * _NSC
RPW = N // NW
NCH = N // 16


def _a_body(fs_ref, ys_ref, fq_ref, dist_ref, a_ref, lam_ref, y_ref,
            q2_ref, q2t_ref):
    eye_b = (lax.broadcasted_iota(jnp.int32, (BR, BR), 0)
             == lax.broadcasted_iota(jnp.int32, (BR, BR), 1)).astype(jnp.float32)

    def q2_blk(bi, _):
        fq = fq_ref[pl.ds(bi * BR, BR), :]
        q2_ref[pl.ds(bi * BR, BR), :] = jnp.sum(fq * fq, axis=1, keepdims=True)
        return 0
    lax.fori_loop(0, NB, q2_blk, 0)

    def t_blk(bi, _):
        q2t_ref[:, pl.ds(bi * BR, BR)] = lax.dot_general(
            q2_ref[pl.ds(bi * BR, BR), :], eye_b, (((0,), (0,)), ((), ())),
            preferred_element_type=jnp.float32, precision=lax.Precision.HIGHEST)
        return 0
    lax.fori_loop(0, NB, t_blk, 0)

    cls_iota = lax.broadcasted_iota(jnp.int32, (C, NS), 0)
    onehot = (cls_iota == ys_ref[...]).astype(jnp.float32)
    counts = jnp.sum(onehot, axis=1, keepdims=True)
    protos = lax.dot_general(
        onehot, fs_ref[...], (((1,), (0,)), ((), ())),
        preferred_element_type=jnp.float32,
        precision=lax.Precision.HIGHEST) / counts

    p2 = jnp.sum(protos * protos, axis=1, keepdims=True)
    eye16 = (lax.broadcasted_iota(jnp.int32, (C, C), 0)
             == lax.broadcasted_iota(jnp.int32, (C, C), 1)).astype(jnp.float32)
    p2t = lax.dot_general(p2, eye16, (((0,), (0,)), ((), ())),
                          preferred_element_type=jnp.float32,
                          precision=lax.Precision.HIGHEST)
    pq = lax.dot_general(fq_ref[...], protos, (((1,), (1,)), ((), ())),
                         preferred_element_type=jnp.float32)
    d2p = jnp.maximum(q2_ref[...] + p2t - 2.0 * pq, 0.0)
    distp = jnp.sqrt(d2p + 1e-12)
    a_ref[...] = distp * distp
    d = jnp.min(distp, axis=1, keepdims=True)

    db = lax.bitcast_convert_type(d, jnp.int32)
    def med_step(_, c):
        lo, hi = c
        mid = lo + (hi - lo) // 2
        cnt = jnp.sum((db <= mid).astype(jnp.int32))
        take = cnt >= (N // 2)
        return jnp.where(take, lo, mid + 1), jnp.where(take, mid, hi)
    lo, _hi = lax.fori_loop(0, 31, med_step, (jnp.int32(0), jnp.int32(2**31 - 1)))
    med = lax.bitcast_convert_type(lo, jnp.float32)
    lam_ref[...] = jnp.exp(-(d * d) / (2.0 * med * med + 1e-8))

    na = -a_ref[...]
    mx = jnp.max(na, axis=1, keepdims=True)
    e = jnp.exp(na - mx)
    y_ref[...] = e / jnp.sum(e, axis=1, keepdims=True)

    def dist_blk(bi, _):
        fq = fq_ref[pl.ds(bi * BR, BR), :]
        g = lax.dot_general(fq, fq_ref[...], (((1,), (1,)), ((), ())),
                            preferred_element_type=jnp.float32)
        d2 = jnp.maximum(q2_ref[pl.ds(bi * BR, BR), :] + q2t_ref[...] - 2.0 * g, 0.0)
        dist_ref[pl.ds(bi * BR, BR), :] = jnp.sqrt(d2 + 1e-12)
        return 0
    lax.fori_loop(0, NB, dist_blk, 0)


def _sc_topk(dist):
    mesh = plsc.VectorSubcoreMesh(core_axis_name="c", subcore_axis_name="s")

    @functools.partial(
        pl.kernel, mesh=mesh,
        out_type=(jax.ShapeDtypeStruct((N * N,), jnp.float32),
                  jax.ShapeDtypeStruct((N * 16,), jnp.float32)),
        scratch_types=[
            pltpu.VMEM((8 * N,), jnp.float32),
            pltpu.VMEM((8 * N,), jnp.float32),
            pltpu.VMEM((NCH,), jnp.float32),
            pltpu.VMEM((8 * 16,), jnp.float32),
        ],
        compiler_params=pltpu.CompilerParams(needs_layout_passes=False),
    )
    def k(dist_hbm, wraw_hbm, sig_hbm, rows_v, lines_v, cm_v, sg_v):
        wid = lax.axis_index("s") * _NC + lax.axis_index("c")
        base = wid * RPW
        lane = lax.iota(jnp.int32, 16)
        lane0 = lane == 0
        zeros16 = jnp.zeros((16,), jnp.float32)
        big16 = jnp.full((16,), BIGF, jnp.float32)

        def batch_body(b, _):
            g0 = base + b * 8
            pltpu.sync_copy(dist_hbm.at[pl.ds(g0 * N, 8 * N)], rows_v)

            def zl(c, _):
                lines_v[pl.ds(c * 16, 16)] = zeros16
                return 0
            lax.fori_loop(0, 8 * NCH, zl, 0)

            def row_body(r8, _):
                ro = r8 * N

                def cmb(c, _):
                    v = rows_v[pl.ds(ro + c * 16, 16)]
                    mn = lax.reduce_min(v, axes=(0,))
                    plsc.store_scatter(cm_v, [jnp.broadcast_to(c, (16,))],
                                       jnp.broadcast_to(mn, (16,)), mask=lane0)
                    return 0
                lax.fori_loop(0, NCH, cmb, 0)

                def round_body(j, _):
                    def gm(cc, acc):
                        return jnp.minimum(acc, cm_v[pl.ds(cc * 16, 16)])
                    accv = lax.fori_loop(0, NCH // 16, gm, big16)
                    mval = lax.reduce_min(accv, axes=(0,))

                    def fc(cc, st):
                        slot, ln = st
                        eq = cm_v[pl.ds(cc * 16, 16)] == mval
                        cand = jnp.where(eq, lane, jnp.int32(16))
                        l_s = lax.reduce_min(cand, axes=(0,))
                        take = (slot < 0) & (l_s < 16)
                        return (jnp.where(take, cc, slot),
                                jnp.where(take, l_s, ln))
                    cslot, clane = lax.fori_loop(
                        0, NCH // 16, fc, (jnp.int32(-1), jnp.int32(0)))
                    ci = cslot * 16 + clane

                    gidx = jnp.broadcast_to(ro + ci * 16, (16,)) + lane
                    v = plsc.load_gather(rows_v, [gidx])
                    candl = jnp.where(v == mval, lane, jnp.int32(16))
                    el = lax.reduce_min(candl, axes=(0,))
                    col = ci * 16 + el

                    plsc.store_scatter(rows_v, [jnp.broadcast_to(ro + col, (16,))],
                                       big16, mask=lane0)
                    v2 = plsc.load_gather(rows_v, [gidx])
                    mn2 = lax.reduce_min(v2, axes=(0,))
                    plsc.store_scatter(cm_v, [jnp.broadcast_to(ci, (16,))],
                                       jnp.broadcast_to(mn2, (16,)), mask=lane0)

                    @pl.when(j > 0)
                    def _():
                        plsc.store_scatter(lines_v,
                                           [jnp.broadcast_to(ro + col, (16,))],
                                           jnp.broadcast_to(mval, (16,)),
                                           mask=lane0)
                    return jnp.broadcast_to(mval, (16,))
                last = lax.fori_loop(0, KP1, round_body, zeros16)

                sg_v[pl.ds(r8 * 16, 16)] = last + 1e-8
                return 0
            lax.fori_loop(0, 8, row_body, 0)

            pltpu.sync_copy(lines_v, wraw_hbm.at[pl.ds(g0 * N, 8 * N)])
            pltpu.sync_copy(sg_v, sig_hbm.at[pl.ds(g0 * 16, 8 * 16)])
            return 0
        lax.fori_loop(0, RPW // 8, batch_body, 0)

    return k(dist)


def _b_body(wraw_ref, sig_ref, a_ref, lam_ref, y0_ref, out_ref,
            wx_ref, sigt_ref, y_ref, coef_ref):
    eye_b = (lax.broadcasted_iota(jnp.int32, (BR, BR), 0)
             == lax.broadcasted_iota(jnp.int32, (BR, BR), 1)).astype(jnp.float32)

    def t_blk(bi, _):
        sigt_ref[:, pl.ds(bi * BR, BR)] = lax.dot_general(
            sig_ref[pl.ds(bi * BR, BR), :], eye_b, (((0,), (0,)), ((), ())),
            preferred_element_type=jnp.float32, precision=lax.Precision.HIGHEST)
        return 0
    lax.fori_loop(0, NB, t_blk, 0)

    y_ref[...] = y0_ref[...]

    def wexp_blk(bi, _):
        sl = pl.ds(bi * BR, BR)
        wv = wraw_ref[sl, :]
        sig = sig_ref[sl, :]
        wx_ref[sl, :] = jnp.where(wv > 0.0,
                                  jnp.exp(-wv / (sig * sigt_ref[...])), 0.0)
        return 0
    lax.fori_loop(0, NB, wexp_blk, 0)

    def w_bi(bi, _):
        def w_bj(bj, _):
            su = pl.ds(bi * BR, BR)
            sv = pl.ds(bj * BR, BR)
            blk_a = wx_ref[su, sv]
            blk_b = wx_ref[sv, su]
            w1 = (blk_a + jnp.transpose(blk_b)) / 2.0
            wx_ref[su, sv] = w1
            wx_ref[sv, su] = jnp.transpose(w1)
            return 0
        lax.fori_loop(bi, NB, w_bj, 0)
        return 0
    lax.fori_loop(0, NB, w_bi, 0)

    def coef_blk(bi, _):
        sl = pl.ds(bi * BR, BR)
        rs = jnp.sum(wx_ref[sl, :], axis=1, keepdims=True)
        d_inv = 1.0 / (rs + 1e-8)
        coef_ref[sl, :] = lam_ref[sl, :] * d_inv
        return 0
    lax.fori_loop(0, NB, coef_blk, 0)

    neg_a = -a_ref[...]
    coef = coef_ref[...]

    def lp_cond(c):
        i, stop = c
        return (i < 50) & jnp.logical_not(stop)

    def lp_body(c):
        i, stop = c
        y_old = y_ref[...]
        z = lax.dot_general(wx_ref[...], y_old, (((1,), (0,)), ((), ())),
                            preferred_element_type=jnp.float32)
        logits = neg_a + coef * z
        mxl = jnp.max(logits, axis=1, keepdims=True)
        el = jnp.exp(logits - mxl)
        y_new = el / jnp.sum(el, axis=1, keepdims=True)
        converged = jnp.max(jnp.abs(y_new - y_old)) < 1e-4

        @pl.when(jnp.logical_not(converged))
        def _():
            y_ref[...] = y_new
        return i + 1, converged

    lax.while_loop(lp_cond, lp_body, (jnp.int32(0), jnp.bool_(False)))

    y = y_ref[...]
    mxy = jnp.max(y, axis=1, keepdims=True)
    cols = lax.broadcasted_iota(jnp.int32, (N, C), 1)
    out_ref[...] = jnp.min(jnp.where(y == mxy, cols, C), axis=1, keepdims=True)


def kernel(feat_s, y_s, feat_q):
    ys2d = y_s.astype(jnp.int32).reshape(1, NS)

    dist, a, lam, y0 = pl.pallas_call(
        _a_body,
        out_shape=(jax.ShapeDtypeStruct((N, N), jnp.float32),
                   jax.ShapeDtypeStruct((N, C), jnp.float32),
                   jax.ShapeDtypeStruct((N, 1), jnp.float32),
                   jax.ShapeDtypeStruct((N, C), jnp.float32)),
        scratch_shapes=[
            pltpu.VMEM((N, 1), jnp.float32),
            pltpu.VMEM((1, N), jnp.float32),
        ],
    )(feat_s, ys2d, feat_q)

    wraw1d, sig1d = _sc_topk(dist.reshape(N * N))
    wraw = wraw1d.reshape(N, N)
    sig = sig1d.reshape(N, 16)[:, 0:1]

    out = pl.pallas_call(
        _b_body,
        out_shape=jax.ShapeDtypeStruct((N, 1), jnp.int32),
        scratch_shapes=[
            pltpu.VMEM((N, N), jnp.float32),
            pltpu.VMEM((1, N), jnp.float32),
            pltpu.VMEM((N, C), jnp.float32),
            pltpu.VMEM((N, 1), jnp.float32),
        ],
    )(wraw, sig, a, lam, y0)
    return out.reshape(N)


# speedup vs baseline: 2.5468x; 2.2776x over previous
import jax
import jax.numpy as jnp
from jax import lax
from jax.experimental import pallas as pl
from jax.experimental.pallas import tpu as pltpu

N = 2048
DIM = 512
NS = 256
C = 16
KP1 = 12
BR = 256
NB = N // BR
BIGF = 3.0e38


def _body(fs_ref, ys_ref, fq_ref, out_ref,
          dist_ref, wv_ref, q2_ref, q2t_ref, sig_ref, sigt_ref,
          a_ref, lam_ref, y_ref, coef_ref):
    eye_b = (lax.broadcasted_iota(jnp.int32, (BR, BR), 0)
             == lax.broadcasted_iota(jnp.int32, (BR, BR), 1)).astype(jnp.float32)

    def col2row(col_ref, row_ref):
        def blk(bi, _):
            row_ref[:, pl.ds(bi * BR, BR)] = lax.dot_general(
                col_ref[pl.ds(bi * BR, BR), :], eye_b, (((0,), (0,)), ((), ())),
                preferred_element_type=jnp.float32, precision=lax.Precision.HIGHEST)
            return 0
        lax.fori_loop(0, NB, blk, 0)

    def q2_blk(bi, _):
        fq = fq_ref[pl.ds(bi * BR, BR), :]
        q2_ref[pl.ds(bi * BR, BR), :] = jnp.sum(fq * fq, axis=1, keepdims=True)
        wv_ref[pl.ds(bi * BR, BR), :] = jnp.zeros((BR, N), jnp.float32)
        return 0
    lax.fori_loop(0, NB, q2_blk, 0)
    col2row(q2_ref, q2t_ref)

    cls_iota = lax.broadcasted_iota(jnp.int32, (C, NS), 0)
    onehot = (cls_iota == ys_ref[...]).astype(jnp.float32)
    counts = jnp.sum(onehot, axis=1, keepdims=True)
    protos = lax.dot_general(
        onehot, fs_ref[...], (((1,), (0,)), ((), ())),
        preferred_element_type=jnp.float32,
        precision=lax.Precision.HIGHEST) / counts

    p2 = jnp.sum(protos * protos, axis=1, keepdims=True)
    eye16 = (lax.broadcasted_iota(jnp.int32, (C, C), 0)
             == lax.broadcasted_iota(jnp.int32, (C, C), 1)).astype(jnp.float32)
    p2t = lax.dot_general(p2, eye16, (((0,), (0,)), ((), ())),
                          preferred_element_type=jnp.float32,
                          precision=lax.Precision.HIGHEST)
    pq = lax.dot_general(fq_ref[...], protos, (((1,), (1,)), ((), ())),
                         preferred_element_type=jnp.float32)
    d2p = jnp.maximum(q2_ref[...] + p2t - 2.0 * pq, 0.0)
    distp = jnp.sqrt(d2p + 1e-12)
    a_ref[...] = distp * distp
    d = jnp.min(distp, axis=1, keepdims=True)

    db = lax.bitcast_convert_type(d, jnp.int32)
    def med_step(_, c):
        lo, hi = c
        mid = lo + (hi - lo) // 2
        cnt = jnp.sum((db <= mid).astype(jnp.int32))
        take = cnt >= (N // 2)
        return jnp.where(take, lo, mid + 1), jnp.where(take, mid, hi)
    lo, _hi = lax.fori_loop(0, 31, med_step, (jnp.int32(0), jnp.int32(2**31 - 1)))
    med = lax.bitcast_convert_type(lo, jnp.float32)
    lam_ref[...] = jnp.exp(-(d * d) / (2.0 * med * med + 1e-8))

    na = -a_ref[...]
    mx = jnp.max(na, axis=1, keepdims=True)
    e = jnp.exp(na - mx)
    y_ref[...] = e / jnp.sum(e, axis=1, keepdims=True)

    def dist_blk(bi, _):
        fq = fq_ref[pl.ds(bi * BR, BR), :]
        g = lax.dot_general(fq, fq_ref[...], (((1,), (1,)), ((), ())),
                            preferred_element_type=jnp.float32)
        d2 = jnp.maximum(q2_ref[pl.ds(bi * BR, BR), :] + q2t_ref[...] - 2.0 * g, 0.0)
        dist_ref[pl.ds(bi * BR, BR), :] = jnp.sqrt(d2 + 1e-12)
        return 0
    lax.fori_loop(0, NB, dist_blk, 0)

    def ext_step(t, _):
        j = t // NB
        bi = t % NB
        sl = pl.ds(bi * BR, BR)
        blk = dist_ref[sl, :]
        m = jnp.min(blk, axis=1, keepdims=True)
        cols = lax.broadcasted_iota(jnp.int32, (BR, N), 1)
        am = jnp.min(jnp.where(blk == m, cols, N), axis=1, keepdims=True)
        chosen = cols == am
        dist_ref[sl, :] = jnp.where(chosen, BIGF, blk)

        @pl.when(j > 0)
        def _():
            wv_ref[sl, :] = jnp.where(chosen, m, wv_ref[sl, :])

        @pl.when(j == KP1 - 1)
        def _():
            sig_ref[sl, :] = m + 1e-8
        return 0
    lax.fori_loop(0, KP1 * NB, ext_step, 0)

    col2row(sig_ref, sigt_ref)

    def wexp_blk(bi, _):
        sl = pl.ds(bi * BR, BR)
        wv = wv_ref[sl, :]
        sig = sig_ref[sl, :]
        wv_ref[sl, :] = jnp.where(wv > 0.0,
                                  jnp.exp(-wv / (sig * sigt_ref[...])), 0.0)
        return 0
    lax.fori_loop(0, NB, wexp_blk, 0)

    def w_blk(bi, _):
        sl = pl.ds(bi * BR, BR)
        def sub(bj, _):
            sc = pl.ds(bj * BR, BR)
            t = jnp.transpose(wv_ref[sc, sl])
            dist_ref[sl, sc] = (wv_ref[sl, sc] + t) / 2.0
            return 0
        lax.fori_loop(0, NB, sub, 0)
        rs = jnp.sum(dist_ref[sl, :], axis=1, keepdims=True)
        d_inv = 1.0 / (rs + 1e-8)
        coef_ref[sl, :] = lam_ref[sl, :] * d_inv
        return 0
    lax.fori_loop(0, NB, w_blk, 0)

    neg_a = -a_ref[...]
    coef = coef_ref[...]

    def lp_cond(c):
        i, stop = c
        return (i < 50) & jnp.logical_not(stop)

    def lp_body(c):
        i, stop = c
        y_old = y_ref[...]
        z = lax.dot_general(dist_ref[...], y_old, (((1,), (0,)), ((), ())),
                            preferred_element_type=jnp.float32)
        logits = neg_a + coef * z
        mxl = jnp.max(logits, axis=1, keepdims=True)
        el = jnp.exp(logits - mxl)
        y_new = el / jnp.sum(el, axis=1, keepdims=True)
        converged = jnp.max(jnp.abs(y_new - y_old)) < 1e-4

        @pl.when(jnp.logical_not(converged))
        def _():
            y_ref[...] = y_new
        return i + 1, converged

    lax.while_loop(lp_cond, lp_body, (jnp.int32(0), jnp.bool_(False)))

    y = y_ref[...]
    mxy = jnp.max(y, axis=1, keepdims=True)
    cols = lax.broadcasted_iota(jnp.int32, (N, C), 1)
    out_ref[...] = jnp.min(jnp.where(y == mxy, cols, C), axis=1, keepdims=True)


def kernel(feat_s, y_s, feat_q):
    ys2d = y_s.astype(jnp.int32).reshape(1, NS)
    out = pl.pallas_call(
        _body,
        out_shape=jax.ShapeDtypeStruct((N, 1), jnp.int32),
        scratch_shapes=[
            pltpu.VMEM((N, N), jnp.float32),
            pltpu.VMEM((N, N), jnp.float32),
            pltpu.VMEM((N, 1), jnp.float32),
            pltpu.VMEM((1, N), jnp.float32),
            pltpu.VMEM((N, 1), jnp.float32),
            pltpu.VMEM((1, N), jnp.float32),
            pltpu.VMEM((N, C), jnp.float32),
            pltpu.VMEM((N, 1), jnp.float32),
            pltpu.VMEM((N, C), jnp.float32),
            pltpu.VMEM((N, 1), jnp.float32),
        ],
    )(feat_s, ys2d, feat_q)
    return out.reshape(N)


# speedup vs baseline: 2.7558x; 1.0820x over previous
import jax
import jax.numpy as jnp
from jax import lax
from jax.experimental import pallas as pl
from jax.experimental.pallas import tpu as pltpu

N = 2048
DIM = 512
NS = 256
C = 16
KP1 = 12
BR = 512
NB = N // BR
BIGF = 3.0e38


def _body(fs_ref, ys_ref, fq_ref, out_ref,
          dist_ref, wv_ref, q2_ref, q2t_ref, sig_ref, sigt_ref,
          a_ref, lam_ref, y_ref, coef_ref):
    eye_b = (lax.broadcasted_iota(jnp.int32, (BR, BR), 0)
             == lax.broadcasted_iota(jnp.int32, (BR, BR), 1)).astype(jnp.float32)

    def col2row(col_ref, row_ref):
        def blk(bi, _):
            row_ref[:, pl.ds(bi * BR, BR)] = lax.dot_general(
                col_ref[pl.ds(bi * BR, BR), :], eye_b, (((0,), (0,)), ((), ())),
                preferred_element_type=jnp.float32, precision=lax.Precision.HIGHEST)
            return 0
        lax.fori_loop(0, NB, blk, 0)

    def q2_blk(bi, _):
        fq = fq_ref[pl.ds(bi * BR, BR), :]
        q2_ref[pl.ds(bi * BR, BR), :] = jnp.sum(fq * fq, axis=1, keepdims=True)
        wv_ref[pl.ds(bi * BR, BR), :] = jnp.zeros((BR, N), jnp.float32)
        return 0
    lax.fori_loop(0, NB, q2_blk, 0)
    col2row(q2_ref, q2t_ref)

    cls_iota = lax.broadcasted_iota(jnp.int32, (C, NS), 0)
    onehot = (cls_iota == ys_ref[...]).astype(jnp.float32)
    counts = jnp.sum(onehot, axis=1, keepdims=True)
    protos = lax.dot_general(
        onehot, fs_ref[...], (((1,), (0,)), ((), ())),
        preferred_element_type=jnp.float32,
        precision=lax.Precision.HIGHEST) / counts

    p2 = jnp.sum(protos * protos, axis=1, keepdims=True)
    eye16 = (lax.broadcasted_iota(jnp.int32, (C, C), 0)
             == lax.broadcasted_iota(jnp.int32, (C, C), 1)).astype(jnp.float32)
    p2t = lax.dot_general(p2, eye16, (((0,), (0,)), ((), ())),
                          preferred_element_type=jnp.float32,
                          precision=lax.Precision.HIGHEST)
    pq = lax.dot_general(fq_ref[...], protos, (((1,), (1,)), ((), ())),
                         preferred_element_type=jnp.float32)
    d2p = jnp.maximum(q2_ref[...] + p2t - 2.0 * pq, 0.0)
    distp = jnp.sqrt(d2p + 1e-12)
    a_ref[...] = distp * distp
    d = jnp.min(distp, axis=1, keepdims=True)

    db = lax.bitcast_convert_type(d, jnp.int32)
    def med_step(_, c):
        lo, hi = c
        mid = lo + (hi - lo) // 2
        cnt = jnp.sum((db <= mid).astype(jnp.int32))
        take = cnt >= (N // 2)
        return jnp.where(take, lo, mid + 1), jnp.where(take, mid, hi)
    lo, _hi = lax.fori_loop(0, 31, med_step, (jnp.int32(0), jnp.int32(2**31 - 1)))
    med = lax.bitcast_convert_type(lo, jnp.float32)
    lam_ref[...] = jnp.exp(-(d * d) / (2.0 * med * med + 1e-8))

    na = -a_ref[...]
    mx = jnp.max(na, axis=1, keepdims=True)
    e = jnp.exp(na - mx)
    y_ref[...] = e / jnp.sum(e, axis=1, keepdims=True)

    def dist_blk(bi, _):
        fq = fq_ref[pl.ds(bi * BR, BR), :]
        g = lax.dot_general(fq, fq_ref[...], (((1,), (1,)), ((), ())),
                            preferred_element_type=jnp.float32)
        d2 = jnp.maximum(q2_ref[pl.ds(bi * BR, BR), :] + q2t_ref[...] - 2.0 * g, 0.0)
        dist_ref[pl.ds(bi * BR, BR), :] = jnp.sqrt(d2 + 1e-12)
        return 0
    lax.fori_loop(0, NB, dist_blk, 0)

    def ext_step(t, _):
        j = t // NB
        bi = t % NB
        sl = pl.ds(bi * BR, BR)
        blk = dist_ref[sl, :]
        m = jnp.min(blk, axis=1, keepdims=True)
        cols = lax.broadcasted_iota(jnp.int32, (BR, N), 1)
        am = jnp.min(jnp.where(blk == m, cols, N), axis=1, keepdims=True)
        chosen = cols == am
        dist_ref[sl, :] = jnp.where(chosen, BIGF, blk)

        @pl.when(j > 0)
        def _():
            wv_ref[sl, :] = jnp.where(chosen, m, wv_ref[sl, :])

        @pl.when(j == KP1 - 1)
        def _():
            sig_ref[sl, :] = m + 1e-8
        return 0
    lax.fori_loop(0, KP1 * NB, ext_step, 0)

    col2row(sig_ref, sigt_ref)

    def wexp_blk(bi, _):
        sl = pl.ds(bi * BR, BR)
        wv = wv_ref[sl, :]
        sig = sig_ref[sl, :]
        wv_ref[sl, :] = jnp.where(wv > 0.0,
                                  jnp.exp(-wv / (sig * sigt_ref[...])), 0.0)
        return 0
    lax.fori_loop(0, NB, wexp_blk, 0)

    def w_blk(bi, _):
        sl = pl.ds(bi * BR, BR)
        def sub(bj, _):
            sc = pl.ds(bj * BR, BR)
            t = jnp.transpose(wv_ref[sc, sl])
            dist_ref[sl, sc] = (wv_ref[sl, sc] + t) / 2.0
            return 0
        lax.fori_loop(0, NB, sub, 0)
        rs = jnp.sum(dist_ref[sl, :], axis=1, keepdims=True)
        d_inv = 1.0 / (rs + 1e-8)
        coef_ref[sl, :] = lam_ref[sl, :] * d_inv
        return 0
    lax.fori_loop(0, NB, w_blk, 0)

    neg_a = -a_ref[...]
    coef = coef_ref[...]

    def lp_cond(c):
        i, stop = c
        return (i < 50) & jnp.logical_not(stop)

    def lp_body(c):
        i, stop = c
        y_old = y_ref[...]
        z = lax.dot_general(dist_ref[...], y_old, (((1,), (0,)), ((), ())),
                            preferred_element_type=jnp.float32)
        logits = neg_a + coef * z
        mxl = jnp.max(logits, axis=1, keepdims=True)
        el = jnp.exp(logits - mxl)
        y_new = el / jnp.sum(el, axis=1, keepdims=True)
        converged = jnp.max(jnp.abs(y_new - y_old)) < 1e-4

        @pl.when(jnp.logical_not(converged))
        def _():
            y_ref[...] = y_new
        return i + 1, converged

    lax.while_loop(lp_cond, lp_body, (jnp.int32(0), jnp.bool_(False)))

    y = y_ref[...]
    mxy = jnp.max(y, axis=1, keepdims=True)
    cols = lax.broadcasted_iota(jnp.int32, (N, C), 1)
    out_ref[...] = jnp.min(jnp.where(y == mxy, cols, C), axis=1, keepdims=True)


def kernel(feat_s, y_s, feat_q):
    ys2d = y_s.astype(jnp.int32).reshape(1, NS)
    out = pl.pallas_call(
        _body,
        out_shape=jax.ShapeDtypeStruct((N, 1), jnp.int32),
        scratch_shapes=[
            pltpu.VMEM((N, N), jnp.float32),
            pltpu.VMEM((N, N), jnp.float32),
            pltpu.VMEM((N, 1), jnp.float32),
            pltpu.VMEM((1, N), jnp.float32),
            pltpu.VMEM((N, 1), jnp.float32),
            pltpu.VMEM((1, N), jnp.float32),
            pltpu.VMEM((N, C), jnp.float32),
            pltpu.VMEM((N, 1), jnp.float32),
            pltpu.VMEM((N, C), jnp.float32),
            pltpu.VMEM((N, 1), jnp.float32),
        ],
    )(feat_s, ys2d, feat_q)
    return out.reshape(N)


# speedup vs baseline: 2.9048x; 1.0541x over previous
import jax
import jax.numpy as jnp
from jax import lax
from jax.experimental import pallas as pl
from jax.experimental.pallas import tpu as pltpu

N = 2048
DIM = 512
NS = 256
C = 16
KP1 = 12
BR = 1024
NB = N // BR
BIGF = 3.0e38


def _body(fs_ref, ys_ref, fq_ref, out_ref,
          dist_ref, wv_ref, q2_ref, q2t_ref, sig_ref, sigt_ref,
          a_ref, lam_ref, y_ref, coef_ref):
    eye_b = (lax.broadcasted_iota(jnp.int32, (BR, BR), 0)
             == lax.broadcasted_iota(jnp.int32, (BR, BR), 1)).astype(jnp.float32)

    def col2row(col_ref, row_ref):
        def blk(bi, _):
            row_ref[:, pl.ds(bi * BR, BR)] = lax.dot_general(
                col_ref[pl.ds(bi * BR, BR), :], eye_b, (((0,), (0,)), ((), ())),
                preferred_element_type=jnp.float32, precision=lax.Precision.HIGHEST)
            return 0
        lax.fori_loop(0, NB, blk, 0)

    def q2_blk(bi, _):
        fq = fq_ref[pl.ds(bi * BR, BR), :]
        q2_ref[pl.ds(bi * BR, BR), :] = jnp.sum(fq * fq, axis=1, keepdims=True)
        wv_ref[pl.ds(bi * BR, BR), :] = jnp.zeros((BR, N), jnp.float32)
        return 0
    lax.fori_loop(0, NB, q2_blk, 0)
    col2row(q2_ref, q2t_ref)

    cls_iota = lax.broadcasted_iota(jnp.int32, (C, NS), 0)
    onehot = (cls_iota == ys_ref[...]).astype(jnp.float32)
    counts = jnp.sum(onehot, axis=1, keepdims=True)
    protos = lax.dot_general(
        onehot, fs_ref[...], (((1,), (0,)), ((), ())),
        preferred_element_type=jnp.float32,
        precision=lax.Precision.HIGHEST) / counts

    p2 = jnp.sum(protos * protos, axis=1, keepdims=True)
    eye16 = (lax.broadcasted_iota(jnp.int32, (C, C), 0)
             == lax.broadcasted_iota(jnp.int32, (C, C), 1)).astype(jnp.float32)
    p2t = lax.dot_general(p2, eye16, (((0,), (0,)), ((), ())),
                          preferred_element_type=jnp.float32,
                          precision=lax.Precision.HIGHEST)
    pq = lax.dot_general(fq_ref[...], protos, (((1,), (1,)), ((), ())),
                         preferred_element_type=jnp.float32)
    d2p = jnp.maximum(q2_ref[...] + p2t - 2.0 * pq, 0.0)
    distp = jnp.sqrt(d2p + 1e-12)
    a_ref[...] = distp * distp
    d = jnp.min(distp, axis=1, keepdims=True)

    db = lax.bitcast_convert_type(d, jnp.int32)
    def med_step(_, c):
        lo, hi = c
        mid = lo + (hi - lo) // 2
        cnt = jnp.sum((db <= mid).astype(jnp.int32))
        take = cnt >= (N // 2)
        return jnp.where(take, lo, mid + 1), jnp.where(take, mid, hi)
    lo, _hi = lax.fori_loop(0, 31, med_step, (jnp.int32(0), jnp.int32(2**31 - 1)))
    med = lax.bitcast_convert_type(lo, jnp.float32)
    lam_ref[...] = jnp.exp(-(d * d) / (2.0 * med * med + 1e-8))

    na = -a_ref[...]
    mx = jnp.max(na, axis=1, keepdims=True)
    e = jnp.exp(na - mx)
    y_ref[...] = e / jnp.sum(e, axis=1, keepdims=True)

    def dist_blk(bi, _):
        fq = fq_ref[pl.ds(bi * BR, BR), :]
        g = lax.dot_general(fq, fq_ref[...], (((1,), (1,)), ((), ())),
                            preferred_element_type=jnp.float32)
        d2 = jnp.maximum(q2_ref[pl.ds(bi * BR, BR), :] + q2t_ref[...] - 2.0 * g, 0.0)
        dist_ref[pl.ds(bi * BR, BR), :] = jnp.sqrt(d2 + 1e-12)
        return 0
    lax.fori_loop(0, NB, dist_blk, 0)

    def ext_step(t, _):
        j = t // NB
        bi = t % NB
        sl = pl.ds(bi * BR, BR)
        blk = dist_ref[sl, :]
        m = jnp.min(blk, axis=1, keepdims=True)
        cols = lax.broadcasted_iota(jnp.int32, (BR, N), 1)
        am = jnp.min(jnp.where(blk == m, cols, N), axis=1, keepdims=True)
        chosen = cols == am
        dist_ref[sl, :] = jnp.where(chosen, BIGF, blk)

        @pl.when(j > 0)
        def _():
            wv_ref[sl, :] = jnp.where(chosen, m, wv_ref[sl, :])

        @pl.when(j == KP1 - 1)
        def _():
            sig_ref[sl, :] = m + 1e-8
        return 0
    lax.fori_loop(0, KP1 * NB, ext_step, 0)

    col2row(sig_ref, sigt_ref)

    def wexp_blk(bi, _):
        sl = pl.ds(bi * BR, BR)
        wv = wv_ref[sl, :]
        sig = sig_ref[sl, :]
        wv_ref[sl, :] = jnp.where(wv > 0.0,
                                  jnp.exp(-wv / (sig * sigt_ref[...])), 0.0)
        return 0
    lax.fori_loop(0, NB, wexp_blk, 0)

    def w_blk(bi, _):
        sl = pl.ds(bi * BR, BR)
        def sub(bj, _):
            sc = pl.ds(bj * BR, BR)
            t = jnp.transpose(wv_ref[sc, sl])
            dist_ref[sl, sc] = (wv_ref[sl, sc] + t) / 2.0
            return 0
        lax.fori_loop(0, NB, sub, 0)
        rs = jnp.sum(dist_ref[sl, :], axis=1, keepdims=True)
        d_inv = 1.0 / (rs + 1e-8)
        coef_ref[sl, :] = lam_ref[sl, :] * d_inv
        return 0
    lax.fori_loop(0, NB, w_blk, 0)

    neg_a = -a_ref[...]
    coef = coef_ref[...]

    def lp_cond(c):
        i, stop = c
        return (i < 50) & jnp.logical_not(stop)

    def lp_body(c):
        i, stop = c
        y_old = y_ref[...]
        z = lax.dot_general(dist_ref[...], y_old, (((1,), (0,)), ((), ())),
                            preferred_element_type=jnp.float32)
        logits = neg_a + coef * z
        mxl = jnp.max(logits, axis=1, keepdims=True)
        el = jnp.exp(logits - mxl)
        y_new = el / jnp.sum(el, axis=1, keepdims=True)
        converged = jnp.max(jnp.abs(y_new - y_old)) < 1e-4

        @pl.when(jnp.logical_not(converged))
        def _():
            y_ref[...] = y_new
        return i + 1, converged

    lax.while_loop(lp_cond, lp_body, (jnp.int32(0), jnp.bool_(False)))

    y = y_ref[...]
    mxy = jnp.max(y, axis=1, keepdims=True)
    cols = lax.broadcasted_iota(jnp.int32, (N, C), 1)
    out_ref[...] = jnp.min(jnp.where(y == mxy, cols, C), axis=1, keepdims=True)


def kernel(feat_s, y_s, feat_q):
    ys2d = y_s.astype(jnp.int32).reshape(1, NS)
    out = pl.pallas_call(
        _body,
        out_shape=jax.ShapeDtypeStruct((N, 1), jnp.int32),
        scratch_shapes=[
            pltpu.VMEM((N, N), jnp.float32),
            pltpu.VMEM((N, N), jnp.float32),
            pltpu.VMEM((N, 1), jnp.float32),
            pltpu.VMEM((1, N), jnp.float32),
            pltpu.VMEM((N, 1), jnp.float32),
            pltpu.VMEM((1, N), jnp.float32),
            pltpu.VMEM((N, C), jnp.float32),
            pltpu.VMEM((N, 1), jnp.float32),
            pltpu.VMEM((N, C), jnp.float32),
            pltpu.VMEM((N, 1), jnp.float32),
        ],
    )(feat_s, ys2d, feat_q)
    return out.reshape(N)
